# baseline (speedup 1.0000x reference)
import jax
import jax.numpy as jnp
from jax.experimental import pallas as pl
from jax.experimental.pallas import tpu as pltpu

N = 512
T = 2
B = 10
F = 64
V = 16
FI = 32
RV = 8
R_MAX = 5.0
CUT = 1000000.0
AVG = 511.0

_TWO_OPI = 0.6366197723675814
_PIO2_HI = 1.57079637050628662109375
_PIO2_LO = -4.37113900018624283e-8


def _sincos(theta):
    q = jnp.round(theta * _TWO_OPI)
    qi = q.astype(jnp.int32)
    y = (theta - q * _PIO2_HI) - q * _PIO2_LO
    y2 = y * y
    ps = 8.3320866e-3 + y2 * -1.9515295e-4
    ps = -0.16665852 + y2 * ps
    sp = y + y * (y2 * ps)
    pc = 4.1619556e-2 + y2 * -1.3585880e-3
    pc = -0.49998520 + y2 * pc
    cp = 1.0 + y2 * pc
    swap = (qi & 1) == 1
    s_neg = (qi & 2) != 0
    c_neg = ((qi + 1) & 2) != 0
    s = jnp.where(swap, cp, sp)
    c = jnp.where(swap, sp, cp)
    s = jnp.where(s_neg, -s, s)
    c = jnp.where(c_neg, -c, c)
    return s, c


def _mace_kernel(x_ref, embed_ref, Wr_s_ref, Wr_v_ref, Wh_ref,
                 Wv_ref, Wsv_ref, Wro_s_ref, Wro_v_ref,
                 vec_out_ref, inv_out_ref,
                 rbcat, ppcat, x4blk, hb, w4, w3):
    f32 = jnp.float32
    bf16 = jnp.bfloat16
    x = x_ref[:]
    ones_row = jnp.ones((1, N), dtype=f32)
    ones_col = jnp.ones((N, 1), dtype=f32)

    rows = jax.lax.broadcasted_iota(jnp.int32, (N, N), 0)
    cols = jax.lax.broadcasted_iota(jnp.int32, (N, N), 1)
    diag = rows == cols
    col_bs = [jnp.dot(x_ref[:, c:c + 1], ones_row, preferred_element_type=f32)
              for c in range(3)]
    row_bs = [jax.lax.dot_general(ones_col, x_ref[:, c:c + 1],
                                  (((1,), (1,)), ((), ())),
                                  preferred_element_type=f32)
              for c in range(3)]
    diffs = [row_bs[c] - col_bs[c] for c in range(3)]
    d2 = (diffs[0] * diffs[0] + diffs[1] * diffs[1]) + diffs[2] * diffs[2]
    d2p = d2 + 1e-9
    invr = jax.lax.rsqrt(d2p)
    r = d2p * invr

    _A = 1.2 * 0.6065306597126334 * 0.6324555320336759
    base = jnp.where(diag, 0.0, _A * invr - (_A * 0.5 / CUT))

    s_cur, c1 = _sincos((jnp.pi / R_MAX) * r)
    c1x2 = 2.0 * c1

    inv_avg = 1.0 / AVG
    h = jnp.broadcast_to(embed_ref[0:1, :], (N, F))
    x41 = jnp.concatenate([x, jnp.ones((N, 1), f32)], axis=1)
    x4blk[:] = jnp.zeros((B * N, 4 * B), dtype=f32)
    agg_s0 = jnp.zeros((N, F), dtype=f32)
    rb_cur = base * s_cur
    rb_prev = jnp.zeros((N, N), dtype=f32)
    pp_cur = rb_cur * invr
    pp_prev = jnp.zeros((N, N), dtype=f32)
    for b in range(B):
        pb = rb_cur.astype(bf16)
        rbcat[:, b * N:(b + 1) * N] = pb
        ppcat[:, b * N:(b + 1) * N] = pp_cur
        x4blk[b * N:(b + 1) * N, 4 * b:4 * b + 4] = x41
        agg_s0 = agg_s0 + jnp.dot(pb, (h * Wr_s_ref[0, b:b + 1, :]).astype(bf16),
                                  preferred_element_type=f32)
        rb_cur, rb_prev = c1x2 * rb_cur - rb_prev, rb_cur
        pp_cur, pp_prev = c1x2 * pp_cur - pp_prev, pp_cur

    yall = jnp.dot(ppcat[:], x4blk[:], preferred_element_type=f32)

    vf = [jnp.zeros((N, V), dtype=f32) for _ in range(3)]
    xcol = [x_ref[:, c:c + 1] for c in range(3)]
    for t in range(T):
        if t == 0:
            agg_s = agg_s0 * inv_avg
        else:
            for b in range(B):
                hb[b * N:(b + 1) * N, :] = (h * Wr_s_ref[t, b:b + 1, :]).astype(bf16)
            agg_s = jnp.dot(rbcat[:], hb[:], preferred_element_type=f32) * inv_avg
        wrvwv = jnp.dot(Wr_v_ref[t], Wv_ref[t], preferred_element_type=f32)
        if t == 0:
            w4[:] = jnp.zeros((4 * B, 4 * V), dtype=f32)
        for k in range(4):
            for b in range(B):
                w4[4 * b + k:4 * b + k + 1, k * V:(k + 1) * V] = wrvwv[b:b + 1, :]
        av = jnp.dot(yall, w4[:], preferred_element_type=f32)
        sv = av[:, 3 * V:4 * V]
        for c in range(3):
            vf[c] = vf[c] + inv_avg * (xcol[c] * sv - av[:, c * V:(c + 1) * V])
        vnorm = vf[0] * vf[0] + vf[1] * vf[1] + vf[2] * vf[2]
        h = jnp.tanh(jnp.dot(agg_s, Wh_ref[t], preferred_element_type=f32)
                     + jnp.dot(vnorm, Wsv_ref[t], preferred_element_type=f32)) + h

    inv_out_ref[:] = jnp.dot(h, Wro_s_ref[:], preferred_element_type=f32)
    w3[:] = jnp.zeros((3 * V, RV * 3), dtype=f32)
    wro = Wro_v_ref[:]
    for c in range(3):
        for rv in range(RV):
            w3[c * V:(c + 1) * V, rv * 3 + c:rv * 3 + c + 1] = wro[:, rv:rv + 1]
    vfcat = jnp.concatenate(vf, axis=1)
    lane = jax.lax.broadcasted_iota(jnp.int32, (1, RV * 3), 1)
    modpat = lane - 3 * jnp.floor(lane.astype(f32) * (1.0 / 3.0)).astype(jnp.int32)
    com = [jnp.sum(x_ref[:, c:c + 1]) * (1.0 / N) for c in range(3)]
    comvec = jnp.where(modpat == 0, com[0],
                       jnp.where(modpat == 1, com[1], com[2]))
    vec_out_ref[:] = (jnp.dot(vfcat, w3[:], preferred_element_type=f32)
                      + comvec)


def kernel(x, embed, Wr_s, Wr_v, Wh, Wv, Wsv, Wro_s, Wro_v):
    f32 = jnp.float32
    vec24, inv = pl.pallas_call(
        _mace_kernel,
        out_shape=(
            jax.ShapeDtypeStruct((N, RV * 3), f32),
            jax.ShapeDtypeStruct((N, FI), f32),
        ),
        in_specs=[pl.BlockSpec(memory_space=pltpu.VMEM) for _ in range(9)],
        out_specs=(pl.BlockSpec(memory_space=pltpu.VMEM),
                   pl.BlockSpec(memory_space=pltpu.VMEM)),
        scratch_shapes=[
            pltpu.VMEM((N, B * N), jnp.bfloat16),
            pltpu.VMEM((N, B * N), f32),
            pltpu.VMEM((B * N, 4 * B), f32),
            pltpu.VMEM((B * N, F), jnp.bfloat16),
            pltpu.VMEM((4 * B, 4 * V), f32),
            pltpu.VMEM((3 * V, RV * 3), f32),
        ],
    )(x, embed, Wr_s, Wr_v, Wh, Wv, Wsv, Wro_s, Wro_v)
    return vec24.reshape(N, RV, 3), inv


# speedup vs baseline: 1.0035x; 1.0035x over previous
import jax
import jax.numpy as jnp
from jax.experimental import pallas as pl
from jax.experimental.pallas import tpu as pltpu

N = 512
T = 2
B = 10
F = 64
V = 16
FI = 32
RV = 8
R_MAX = 5.0
CUT = 1000000.0
AVG = 511.0

_TWO_OPI = 0.6366197723675814
_PIO2_HI = 1.57079637050628662109375
_PIO2_LO = -4.37113900018624283e-8


def _sincos(theta):
    q = jnp.round(theta * _TWO_OPI)
    qi = q.astype(jnp.int32)
    y = (theta - q * _PIO2_HI) - q * _PIO2_LO
    y2 = y * y
    ps = 8.3320866e-3 + y2 * -1.9515295e-4
    ps = -0.16665852 + y2 * ps
    sp = y + y * (y2 * ps)
    pc = 4.1619556e-2 + y2 * -1.3585880e-3
    pc = -0.49998520 + y2 * pc
    cp = 1.0 + y2 * pc
    swap = (qi & 1) == 1
    s_neg = (qi & 2) != 0
    c_neg = ((qi + 1) & 2) != 0
    s = jnp.where(swap, cp, sp)
    c = jnp.where(swap, sp, cp)
    s = jnp.where(s_neg, -s, s)
    c = jnp.where(c_neg, -c, c)
    return s, c


def _mace_kernel(x_ref, embed_ref, Wr_s_ref, Wr_v_ref, Wh_ref,
                 Wv_ref, Wsv_ref, Wro_s_ref, Wro_v_ref,
                 vec_out_ref, inv_out_ref,
                 rbcat, ppcat, x4blk, hb, w4, w3):
    f32 = jnp.float32
    bf16 = jnp.bfloat16
    x = x_ref[:]
    ones_row = jnp.ones((1, N), dtype=f32)
    ones_col = jnp.ones((N, 1), dtype=f32)

    rows = jax.lax.broadcasted_iota(jnp.int32, (N, N), 0)
    cols = jax.lax.broadcasted_iota(jnp.int32, (N, N), 1)
    diag = rows == cols
    col_bs = [jnp.dot(x_ref[:, c:c + 1], ones_row, preferred_element_type=f32)
              for c in range(3)]
    row_bs = [jax.lax.dot_general(ones_col, x_ref[:, c:c + 1],
                                  (((1,), (1,)), ((), ())),
                                  preferred_element_type=f32)
              for c in range(3)]
    diffs = [row_bs[c] - col_bs[c] for c in range(3)]
    d2 = (diffs[0] * diffs[0] + diffs[1] * diffs[1]) + diffs[2] * diffs[2]
    d2p = d2 + 1e-9
    invr = jax.lax.rsqrt(d2p)
    r = d2p * invr

    _A = 1.2 * 0.6065306597126334 * 0.6324555320336759
    base = jnp.where(diag, 0.0, _A * invr - (_A * 0.5 / CUT))

    s_cur, c1 = _sincos((jnp.pi / R_MAX) * r)
    c1x2 = 2.0 * c1

    inv_avg = 1.0 / AVG
    h = jnp.broadcast_to(embed_ref[0:1, :], (N, F))
    x41 = jnp.concatenate([x, jnp.ones((N, 1), f32)], axis=1)
    x4blk[:] = jnp.zeros((B * N, 4 * B), dtype=f32)
    agg_s0 = jnp.zeros((N, F), dtype=f32)
    rb_cur = base * s_cur
    rb_prev = jnp.zeros((N, N), dtype=f32)
    pp_cur = rb_cur * invr
    pp_prev = jnp.zeros((N, N), dtype=f32)
    for b in range(B):
        pb = rb_cur
        rbcat[:, b * N:(b + 1) * N] = pb
        ppcat[:, b * N:(b + 1) * N] = pp_cur
        x4blk[b * N:(b + 1) * N, 4 * b:4 * b + 4] = x41
        agg_s0 = agg_s0 + jnp.dot(pb, (h * Wr_s_ref[0, b:b + 1, :]),
                                  preferred_element_type=f32)
        rb_cur, rb_prev = c1x2 * rb_cur - rb_prev, rb_cur
        pp_cur, pp_prev = c1x2 * pp_cur - pp_prev, pp_cur

    yall = jnp.dot(ppcat[:], x4blk[:], preferred_element_type=f32)

    vf = [jnp.zeros((N, V), dtype=f32) for _ in range(3)]
    xcol = [x_ref[:, c:c + 1] for c in range(3)]
    for t in range(T):
        if t == 0:
            agg_s = agg_s0 * inv_avg
        else:
            for b in range(B):
                hb[b * N:(b + 1) * N, :] = (h * Wr_s_ref[t, b:b + 1, :])
            agg_s = jnp.dot(rbcat[:], hb[:], preferred_element_type=f32) * inv_avg
        wrvwv = jnp.dot(Wr_v_ref[t], Wv_ref[t], preferred_element_type=f32)
        if t == 0:
            w4[:] = jnp.zeros((4 * B, 4 * V), dtype=f32)
        for k in range(4):
            for b in range(B):
                w4[4 * b + k:4 * b + k + 1, k * V:(k + 1) * V] = wrvwv[b:b + 1, :]
        av = jnp.dot(yall, w4[:], preferred_element_type=f32)
        sv = av[:, 3 * V:4 * V]
        for c in range(3):
            vf[c] = vf[c] + inv_avg * (xcol[c] * sv - av[:, c * V:(c + 1) * V])
        vnorm = vf[0] * vf[0] + vf[1] * vf[1] + vf[2] * vf[2]
        h = jnp.tanh(jnp.dot(agg_s, Wh_ref[t], preferred_element_type=f32)
                     + jnp.dot(vnorm, Wsv_ref[t], preferred_element_type=f32)) + h

    inv_out_ref[:] = jnp.dot(h, Wro_s_ref[:], preferred_element_type=f32)
    w3[:] = jnp.zeros((3 * V, RV * 3), dtype=f32)
    wro = Wro_v_ref[:]
    for c in range(3):
        for rv in range(RV):
            w3[c * V:(c + 1) * V, rv * 3 + c:rv * 3 + c + 1] = wro[:, rv:rv + 1]
    vfcat = jnp.concatenate(vf, axis=1)
    lane = jax.lax.broadcasted_iota(jnp.int32, (1, RV * 3), 1)
    modpat = lane - 3 * jnp.floor(lane.astype(f32) * (1.0 / 3.0)).astype(jnp.int32)
    com = [jnp.sum(x_ref[:, c:c + 1]) * (1.0 / N) for c in range(3)]
    comvec = jnp.where(modpat == 0, com[0],
                       jnp.where(modpat == 1, com[1], com[2]))
    vec_out_ref[:] = (jnp.dot(vfcat, w3[:], preferred_element_type=f32)
                      + comvec)


def kernel(x, embed, Wr_s, Wr_v, Wh, Wv, Wsv, Wro_s, Wro_v):
    f32 = jnp.float32
    vec24, inv = pl.pallas_call(
        _mace_kernel,
        out_shape=(
            jax.ShapeDtypeStruct((N, RV * 3), f32),
            jax.ShapeDtypeStruct((N, FI), f32),
        ),
        in_specs=[pl.BlockSpec(memory_space=pltpu.VMEM) for _ in range(9)],
        out_specs=(pl.BlockSpec(memory_space=pltpu.VMEM),
                   pl.BlockSpec(memory_space=pltpu.VMEM)),
        scratch_shapes=[
            pltpu.VMEM((N, B * N), f32),
            pltpu.VMEM((N, B * N), f32),
            pltpu.VMEM((B * N, 4 * B), f32),
            pltpu.VMEM((B * N, F), f32),
            pltpu.VMEM((4 * B, 4 * V), f32),
            pltpu.VMEM((3 * V, RV * 3), f32),
        ],
    )(x, embed, Wr_s, Wr_v, Wh, Wv, Wsv, Wro_s, Wro_v)
    return vec24.reshape(N, RV, 3), inv
